# baseline (speedup 1.0000x reference)
import functools

import jax
import jax.numpy as jnp
from jax import lax
from jax.experimental import pallas as pl
from jax.experimental.pallas import tpu as pltpu
from jax.experimental.pallas import tpu_sc as plsc

_MASK_RATIO = 0.25
_NC = 2
_NS = 16


def _msk_body(x_ref, msk_ref, mask_ref, *, nv):
    i = pl.program_id(0)
    j = pl.program_id(1)

    @pl.when(jnp.logical_and(i == 0, j == 0))
    def _():
        b, s = mask_ref.shape
        col = jax.lax.broadcasted_iota(jnp.int32, (b, s), 1)
        mask_ref[...] = col >= nv

    msk_ref[...] = x_ref[...]


def _make_sc_copy(b, s, f, row0, rows, dtype):
    nw = _NC * _NS
    rpw = (b * rows) // nw
    wpb = rows // rpw
    ch = min(16, rpw)
    nch = rpw // ch

    mesh = plsc.VectorSubcoreMesh(core_axis_name="c", subcore_axis_name="s")

    @functools.partial(
        pl.kernel,
        mesh=mesh,
        out_type=jax.ShapeDtypeStruct((b, rows, f), dtype),
        scratch_types=[
            pltpu.VMEM((ch, f), jnp.float32),
            pltpu.VMEM((ch, f), jnp.float32),
            pltpu.SemaphoreType.DMA,
            pltpu.SemaphoreType.DMA,
            pltpu.SemaphoreType.DMA,
            pltpu.SemaphoreType.DMA,
        ],
        cost_estimate=pl.CostEstimate(
            flops=0, bytes_accessed=2 * b * rows * f * 4, transcendentals=0
        ),
    )
    def sc_copy(x_hbm, out_hbm, buf0, buf1, si0, si1, so0, so1):
        wid = lax.axis_index("s") * _NC + lax.axis_index("c")
        bi = wid // wpb
        k = wid % wpb
        src0 = row0 + k * rpw
        dst0 = k * rpw

        bufs = (buf0, buf1)
        sin = (si0, si1)
        sout = (so0, so1)
        cin = [None, None]
        cout = [None, None]
        cin[0] = pltpu.async_copy(x_hbm.at[bi, pl.ds(src0, ch)], buf0, si0)
        for c in range(nch):
            p = c % 2
            cin[p].wait()
            if c + 1 < nch:
                q = (c + 1) % 2
                if cout[q] is not None:
                    cout[q].wait()
                cin[q] = pltpu.async_copy(
                    x_hbm.at[bi, pl.ds(src0 + (c + 1) * ch, ch)], bufs[q], sin[q]
                )
            cout[p] = pltpu.async_copy(
                bufs[p], out_hbm.at[bi, pl.ds(dst0 + c * ch, ch)], sout[p]
            )
        cout[(nch - 1) % 2].wait()
        if nch > 1:
            cout[nch % 2].wait()

    return sc_copy


def kernel(x):
    b, s, f = x.shape
    num_mask = int(s * _MASK_RATIO)
    nv = s - num_mask

    bs = 1024
    nvb = nv // bs
    masked, mask = pl.pallas_call(
        functools.partial(_msk_body, nv=nv),
        grid=(b, num_mask // bs),
        in_specs=[pl.BlockSpec((1, bs, f), lambda i, j, nvb=nvb: (i, j + nvb, 0))],
        out_specs=[
            pl.BlockSpec((1, bs, f), lambda i, j: (i, j, 0)),
            pl.BlockSpec((b, s), lambda i, j: (0, 0)),
        ],
        out_shape=[
            jax.ShapeDtypeStruct((b, num_mask, f), x.dtype),
            jax.ShapeDtypeStruct((b, s), jnp.bool_),
        ],
        cost_estimate=pl.CostEstimate(
            flops=0, bytes_accessed=2 * b * num_mask * f * 4, transcendentals=0
        ),
    )(x)

    visible = _make_sc_copy(b, s, f, 0, nv, x.dtype)(x)

    return visible, masked, mask


# speedup vs baseline: 1.0413x; 1.0413x over previous
import functools

import jax
import jax.numpy as jnp
from jax import lax
from jax.experimental import pallas as pl
from jax.experimental.pallas import tpu as pltpu
from jax.experimental.pallas import tpu_sc as plsc

_MASK_RATIO = 0.25
_NC = 2
_NS = 16


def _vis_body(x_ref, vis_ref, mask_ref, *, nv):
    i = pl.program_id(0)
    j = pl.program_id(1)

    @pl.when(jnp.logical_and(i == 0, j == 0))
    def _():
        b, s = mask_ref.shape
        col = jax.lax.broadcasted_iota(jnp.int32, (b, s), 1)
        mask_ref[...] = col >= nv

    vis_ref[...] = x_ref[...]


def _make_sc_copy(b, s, f, row0, rows, dtype, ch=16, nbuf=3):
    nw = _NC * _NS
    rpw = (b * rows) // nw
    wpb = rows // rpw
    ch = min(ch, rpw)
    nch = rpw // ch

    mesh = plsc.VectorSubcoreMesh(core_axis_name="c", subcore_axis_name="s")

    @functools.partial(
        pl.kernel,
        mesh=mesh,
        out_type=jax.ShapeDtypeStruct((b, rows, f), dtype),
        scratch_types=(
            [pltpu.VMEM((ch, f), jnp.float32) for _ in range(nbuf)]
            + [pltpu.SemaphoreType.DMA for _ in range(2 * nbuf)]
        ),
        cost_estimate=pl.CostEstimate(
            flops=0, bytes_accessed=2 * b * rows * f * 4, transcendentals=0
        ),
    )
    def sc_copy(x_hbm, out_hbm, *scratch):
        bufs = scratch[:nbuf]
        sin = scratch[nbuf:2 * nbuf]
        sout = scratch[2 * nbuf:]
        wid = lax.axis_index("s") * _NC + lax.axis_index("c")
        bi = wid // wpb
        k = wid % wpb
        src0 = row0 + k * rpw
        dst0 = k * rpw

        def start_in(c, p):
            return pltpu.async_copy(
                x_hbm.at[bi, pl.ds(src0 + c * ch, ch)], bufs[p], sin[p]
            )

        def start_out(c, p):
            return pltpu.async_copy(
                bufs[p], out_hbm.at[bi, pl.ds(dst0 + c * ch, ch)], sout[p]
            )

        cin = [None] * nbuf
        cout = [None] * nbuf
        for c in range(min(nbuf - 1, nch)):
            cin[c] = start_in(c, c)
        for c in range(nch):
            p = c % nbuf
            cin[p].wait()
            cout[p] = start_out(c, p)
            nxt = c + nbuf - 1
            if nxt < nch:
                q = nxt % nbuf
                if cout[q] is not None:
                    cout[q].wait()
                cin[q] = start_in(nxt, q)
        for c in range(max(0, nch - nbuf), nch):
            cout[c % nbuf].wait()

    return sc_copy


def kernel(x):
    b, s, f = x.shape
    num_mask = int(s * _MASK_RATIO)
    nv = s - num_mask

    bs = 1024
    visible, mask = pl.pallas_call(
        functools.partial(_vis_body, nv=nv),
        grid=(b, nv // bs),
        in_specs=[pl.BlockSpec((1, bs, f), lambda i, j: (i, j, 0))],
        out_specs=[
            pl.BlockSpec((1, bs, f), lambda i, j: (i, j, 0)),
            pl.BlockSpec((b, s), lambda i, j: (0, 0)),
        ],
        out_shape=[
            jax.ShapeDtypeStruct((b, nv, f), x.dtype),
            jax.ShapeDtypeStruct((b, s), jnp.bool_),
        ],
        cost_estimate=pl.CostEstimate(
            flops=0, bytes_accessed=2 * b * nv * f * 4, transcendentals=0
        ),
    )(x)

    masked = _make_sc_copy(b, s, f, nv, num_mask, x.dtype)(x)

    return visible, masked, mask


# speedup vs baseline: 1.0524x; 1.0107x over previous
import functools

import jax
import jax.numpy as jnp
from jax import lax
from jax.experimental import pallas as pl
from jax.experimental.pallas import tpu as pltpu
from jax.experimental.pallas import tpu_sc as plsc

_MASK_RATIO = 0.25
_NC = 2
_NS = 16


def _vis_body(x_ref, vis_ref, mask_ref, *, nv):
    i = pl.program_id(0)
    j = pl.program_id(1)

    @pl.when(jnp.logical_and(i == 0, j == 0))
    def _():
        b, s = mask_ref.shape
        col = jax.lax.broadcasted_iota(jnp.int32, (b, s), 1)
        mask_ref[...] = col >= nv

    vis_ref[...] = x_ref[...]


def _make_sc_copy(b, s, f, row0, rows, dtype, ch=16, nbuf=3):
    nw = _NC * _NS
    rpw = (b * rows) // nw
    wpb = rows // rpw
    ch = min(ch, rpw)
    nch = rpw // ch

    mesh = plsc.VectorSubcoreMesh(core_axis_name="c", subcore_axis_name="s")

    @functools.partial(
        pl.kernel,
        mesh=mesh,
        out_type=jax.ShapeDtypeStruct((b, rows, f), dtype),
        scratch_types=(
            [pltpu.VMEM((ch, f), jnp.float32) for _ in range(nbuf)]
            + [pltpu.SemaphoreType.DMA for _ in range(2 * nbuf)]
        ),
        cost_estimate=pl.CostEstimate(
            flops=0, bytes_accessed=2 * b * rows * f * 4, transcendentals=0
        ),
    )
    def sc_copy(x_hbm, out_hbm, *scratch):
        bufs = scratch[:nbuf]
        sin = scratch[nbuf:2 * nbuf]
        sout = scratch[2 * nbuf:]
        wid = lax.axis_index("s") * _NC + lax.axis_index("c")
        bi = wid // wpb
        k = wid % wpb
        src0 = row0 + k * rpw
        dst0 = k * rpw

        def start_in(c, p):
            return pltpu.async_copy(
                x_hbm.at[bi, pl.ds(src0 + c * ch, ch)], bufs[p], sin[p]
            )

        def start_out(c, p):
            return pltpu.async_copy(
                bufs[p], out_hbm.at[bi, pl.ds(dst0 + c * ch, ch)], sout[p]
            )

        cin = [None] * nbuf
        cout = [None] * nbuf
        for c in range(min(nbuf - 1, nch)):
            cin[c] = start_in(c, c)
        for c in range(nch):
            p = c % nbuf
            cin[p].wait()
            cout[p] = start_out(c, p)
            nxt = c + nbuf - 1
            if nxt < nch:
                q = nxt % nbuf
                if cout[q] is not None:
                    cout[q].wait()
                cin[q] = start_in(nxt, q)
        for c in range(max(0, nch - nbuf), nch):
            cout[c % nbuf].wait()

    return sc_copy


def kernel(x):
    b, s, f = x.shape
    num_mask = int(s * _MASK_RATIO)
    nv = s - num_mask

    bs = 1536
    visible, mask = pl.pallas_call(
        functools.partial(_vis_body, nv=nv),
        grid=(b, nv // bs),
        in_specs=[pl.BlockSpec((1, bs, f), lambda i, j: (i, j, 0))],
        out_specs=[
            pl.BlockSpec((1, bs, f), lambda i, j: (i, j, 0)),
            pl.BlockSpec((b, s), lambda i, j: (0, 0)),
        ],
        out_shape=[
            jax.ShapeDtypeStruct((b, nv, f), x.dtype),
            jax.ShapeDtypeStruct((b, s), jnp.bool_),
        ],
        cost_estimate=pl.CostEstimate(
            flops=0, bytes_accessed=2 * b * nv * f * 4, transcendentals=0
        ),
    )(x)

    masked = _make_sc_copy(b, s, f, nv, num_mask, x.dtype)(x)

    return visible, masked, mask
